# Initial kernel scaffold; baseline (speedup 1.0000x reference)
import functools

import jax
import jax.numpy as jnp
from jax import lax
from jax.experimental import pallas as pl
from jax.experimental.pallas import tpu as pltpu
from jax.experimental.pallas import tpu_sc as plsc

B, H, KVH, Dh = 16, 32, 8, 128
NUM_SLOTS, L = 32768, 2048
SCALE = 0.08838834764831845
GROUP = H // KVH
D = KVH * Dh
NEG = jnp.float32(-1e30)

NC, NS = 2, 16
NW = NC * NS
CH = 24
MAXCH = (L // 2 + CH - 1) // CH + 1

L_BLK = 256
NBLK = L // L_BLK


def _sc_gather_body(kc_hbm, vc_hbm, af_hbm, ctx_hbm, kg_hbm, vg_hbm,
                    idx_v, ctx_v, kbuf, vbuf, gsem, wsem0, wsem1):
    cid = lax.axis_index("c")
    sid = lax.axis_index("s")
    wid = sid * NC + cid
    b = wid // 2
    j = wid % 2

    pltpu.sync_copy(ctx_hbm, ctx_v)
    ctxv = ctx_v[...]
    ctx = jnp.max(jnp.where(lax.iota(jnp.int32, NS) == b, ctxv, 0))

    half = jnp.minimum(((ctx + 1) // 2 + 7) // 8 * 8, L // 2)
    lo = j * half
    hi = jnp.where(j == 0, half, ctx)

    pltpu.sync_copy(af_hbm.at[pl.ds(b * L, L)], idx_v)

    row0 = b * L

    def chunk_base(i):
        return jnp.minimum(lo + i * CH, L - CH)

    def body(i, _):
        s = i % 2
        pred = lo + i * CH < hi
        prevp = (i >= 2) & (lo + (i - 2) * CH < hi)

        @pl.when(prevp)
        def _wait_prev_wb():
            @pl.when(s == 0)
            def _():
                pltpu.make_async_copy(kbuf.at[0], kg_hbm.at[pl.ds(row0, CH)], wsem0).wait()
                pltpu.make_async_copy(vbuf.at[0], vg_hbm.at[pl.ds(row0, CH)], wsem0).wait()
            @pl.when(s == 1)
            def _():
                pltpu.make_async_copy(kbuf.at[1], kg_hbm.at[pl.ds(row0, CH)], wsem1).wait()
                pltpu.make_async_copy(vbuf.at[1], vg_hbm.at[pl.ds(row0, CH)], wsem1).wait()

        @pl.when(pred)
        def _do_chunk():
            base = chunk_base(i)
            idx = idx_v.at[pl.ds(base, CH)]
            pltpu.make_async_copy(kc_hbm.at[idx], kbuf.at[s], gsem).start()
            pltpu.make_async_copy(vc_hbm.at[idx], vbuf.at[s], gsem).start()
            pltpu.make_async_copy(kc_hbm.at[idx], kbuf.at[s], gsem).wait()
            pltpu.make_async_copy(vc_hbm.at[idx], vbuf.at[s], gsem).wait()
            dstk = kg_hbm.at[pl.ds(row0 + base, CH)]
            dstv = vg_hbm.at[pl.ds(row0 + base, CH)]

            @pl.when(s == 0)
            def _():
                pltpu.make_async_copy(kbuf.at[0], dstk, wsem0).start()
                pltpu.make_async_copy(vbuf.at[0], dstv, wsem0).start()
            @pl.when(s == 1)
            def _():
                pltpu.make_async_copy(kbuf.at[1], dstk, wsem1).start()
                pltpu.make_async_copy(vbuf.at[1], dstv, wsem1).start()
        return 0

    lax.fori_loop(0, MAXCH, body, 0)

    nv = jnp.maximum((hi - lo + CH - 1) // CH, 0)

    def drain(slot_parity):
        @pl.when(slot_parity == 0)
        def _():
            pltpu.make_async_copy(kbuf.at[0], kg_hbm.at[pl.ds(row0, CH)], wsem0).wait()
            pltpu.make_async_copy(vbuf.at[0], vg_hbm.at[pl.ds(row0, CH)], wsem0).wait()
        @pl.when(slot_parity == 1)
        def _():
            pltpu.make_async_copy(kbuf.at[1], kg_hbm.at[pl.ds(row0, CH)], wsem1).wait()
            pltpu.make_async_copy(vbuf.at[1], vg_hbm.at[pl.ds(row0, CH)], wsem1).wait()

    @pl.when(nv >= 2)
    def _():
        drain((nv - 2) % 2)

    @pl.when(nv >= 1)
    def _():
        drain((nv - 1) % 2)


def _sc_gather(kc2, vc2, af, context_lens):
    fn = pl.kernel(
        _sc_gather_body,
        out_type=(jax.ShapeDtypeStruct((B * L, D), jnp.float32),
                  jax.ShapeDtypeStruct((B * L, D), jnp.float32)),
        mesh=plsc.VectorSubcoreMesh(core_axis_name="c", subcore_axis_name="s",
                                    num_cores=NC, num_subcores=NS),
        scratch_types=[
            pltpu.VMEM((L,), jnp.int32),
            pltpu.VMEM((NS,), jnp.int32),
            pltpu.VMEM((2, CH, D), jnp.float32),
            pltpu.VMEM((2, CH, D), jnp.float32),
            pltpu.SemaphoreType.DMA,
            pltpu.SemaphoreType.DMA,
            pltpu.SemaphoreType.DMA,
        ],
    )
    return fn(kc2, vc2, af, context_lens)


def _attn_body(ctx_ref, q_ref, ids_ref, sm_ref, knew_ref, vnew_ref,
               kg_ref, vg_ref, o_ref, m_scr, s_scr, acc_scr):
    b = pl.program_id(0)
    c = pl.program_id(1)
    ctx = ctx_ref[b]
    nlast = (ctx + L_BLK - 1) // L_BLK - 1

    @pl.when(c == 0)
    def _init():
        m_scr[...] = jnp.full((H, 128), NEG, jnp.float32)
        s_scr[...] = jnp.zeros((H, 128), jnp.float32)
        acc_scr[...] = jnp.zeros((H, D), jnp.float32)

    @pl.when(c <= nlast)
    def _compute():
        ids = ids_ref[0, 0]
        smv = sm_ref[...]
        onehot_t = (smv == ids).astype(jnp.float32)
        dn = (((0,), (0,)), ((), ()))
        repl_k = lax.dot_general(onehot_t, knew_ref[...], dn,
                                 preferred_element_type=jnp.float32)
        repl_v = lax.dot_general(onehot_t, vnew_ref[...], dn,
                                 preferred_element_type=jnp.float32)
        anym = lax.dot_general(onehot_t, jnp.ones((16, 1), jnp.float32), dn,
                               preferred_element_type=jnp.float32)
        liota = lax.broadcasted_iota(jnp.int32, (L_BLK, 1), 0) + c * L_BLK
        kf = kg_ref[0, 0] * (1.0 - anym) + repl_k
        vf = vg_ref[0, 0] * (1.0 - anym) + repl_v
        vf = jnp.where(liota < ctx, vf, 0.0)

        qb = q_ref[0]
        logits = lax.dot_general(qb, kf, (((1,), (1,)), ((), ())),
                                 preferred_element_type=jnp.float32) * SCALE
        cmask = lax.broadcasted_iota(jnp.int32, (1, L_BLK), 1) + c * L_BLK < ctx
        logits = jnp.where(cmask, logits, NEG)

        m_prev = m_scr[:, :1]
        m_new = jnp.maximum(m_prev, jnp.max(logits, axis=1, keepdims=True))
        alpha = jnp.exp(m_prev - m_new)
        p = jnp.exp(logits - m_new)
        s_new = s_scr[:, :1] * alpha + jnp.sum(p, axis=1, keepdims=True)
        acc_scr[...] = acc_scr[...] * alpha + lax.dot_general(
            p, vf, (((1,), (0,)), ((), ())), preferred_element_type=jnp.float32)
        m_scr[...] = jnp.broadcast_to(m_new, (H, 128))
        s_scr[...] = jnp.broadcast_to(s_new, (H, 128))

    @pl.when(c == NBLK - 1)
    def _fin():
        accv = acc_scr[...] / s_scr[:, :1]
        rowh = lax.broadcasted_iota(jnp.int32, (H, 1), 0) // GROUP
        o = jnp.zeros((H, Dh), jnp.float32)
        for hh in range(KVH):
            o = o + jnp.where(rowh == hh, accv[:, hh * Dh:(hh + 1) * Dh], 0.0)
        o_ref[0] = o


def _ceff(c, ctx):
    return jnp.minimum(c, jnp.maximum((ctx + L_BLK - 1) // L_BLK - 1, 0))


def _attn(context_lens, q_bd, active4, sm2, knew, vnew, kg4, vg4):
    grid_spec = pltpu.PrefetchScalarGridSpec(
        num_scalar_prefetch=1,
        grid=(B, NBLK),
        in_specs=[
            pl.BlockSpec((1, H, D), lambda b, c, ctx: (b, 0, 0)),
            pl.BlockSpec((1, 1, 1, L_BLK),
                         lambda b, c, ctx: (b, _ceff(c, ctx[b]), 0, 0)),
            pl.BlockSpec((16, 1), lambda b, c, ctx: (0, 0)),
            pl.BlockSpec((16, D), lambda b, c, ctx: (0, 0)),
            pl.BlockSpec((16, D), lambda b, c, ctx: (0, 0)),
            pl.BlockSpec((1, 1, L_BLK, D),
                         lambda b, c, ctx: (b, _ceff(c, ctx[b]), 0, 0)),
            pl.BlockSpec((1, 1, L_BLK, D),
                         lambda b, c, ctx: (b, _ceff(c, ctx[b]), 0, 0)),
        ],
        out_specs=pl.BlockSpec((1, H, Dh), lambda b, c, ctx: (b, 0, 0)),
        scratch_shapes=[
            pltpu.VMEM((H, 128), jnp.float32),
            pltpu.VMEM((H, 128), jnp.float32),
            pltpu.VMEM((H, D), jnp.float32),
        ],
    )
    return pl.pallas_call(
        _attn_body,
        grid_spec=grid_spec,
        out_shape=jax.ShapeDtypeStruct((B, H, Dh), jnp.float32),
        compiler_params=pltpu.CompilerParams(
            dimension_semantics=("arbitrary", "arbitrary")),
    )(context_lens, q_bd, active4, sm2, knew, vnew, kg4, vg4)


def _build_q_bd(q):
    q_tiled = jnp.tile(q, (1, 1, KVH))
    rowh = jnp.arange(H) // GROUP
    colh = jnp.arange(D) // Dh
    mask = (rowh[:, None] == colh[None, :]).astype(q.dtype)
    return q_tiled * mask[None]


def kernel(q, k, v, k_cache, v_cache, slot_mapping, active_slots, context_lens):
    kc2 = k_cache.reshape(NUM_SLOTS, D)
    vc2 = v_cache.reshape(NUM_SLOTS, D)
    af = active_slots.reshape(B * L)
    kg, vg = _sc_gather(kc2, vc2, af, context_lens)

    q_bd = _build_q_bd(q)
    active4 = active_slots.reshape(B, NBLK, 1, L_BLK)
    sm2 = slot_mapping.reshape(16, 1)
    knew = k.reshape(B, D)
    vnew = v.reshape(B, D)
    kg4 = kg.reshape(B, NBLK, L_BLK, D)
    vg4 = vg.reshape(B, NBLK, L_BLK, D)
    return _attn(context_lens, q_bd, active4, sm2, knew, vnew, kg4, vg4)


# speedup vs baseline: 2.6228x; 2.6228x over previous
import functools

import jax
import jax.numpy as jnp
from jax import lax
from jax.experimental import pallas as pl
from jax.experimental.pallas import tpu as pltpu
from jax.experimental.pallas import tpu_sc as plsc

B, H, KVH, Dh = 16, 32, 8, 128
NUM_SLOTS, L = 32768, 2048
SCALE = 0.08838834764831845
GROUP = H // KVH
D = KVH * Dh
NEG = -1e30

NC, NS = 2, 16
NW = NC * NS
CH = 24
MAXCH = (L // 2 + CH - 1) // CH + 1

L_BLK = 256
NBLK = L // L_BLK


def _sc_gather_body(kc_hbm, vc_hbm, af_hbm, ctx_hbm, kg_hbm, vg_hbm,
                    idx_v, ctx_v, kbuf, vbuf, gsem, wsem0, wsem1):
    cid = lax.axis_index("c")
    sid = lax.axis_index("s")
    wid = sid * NC + cid
    b = wid // 2
    j = wid % 2

    pltpu.sync_copy(ctx_hbm, ctx_v.at[pl.ds(0, NS)])
    ctx = ctx_v[pl.ds(b, NS)][0]

    half = jnp.minimum(((ctx + 1) // 2 + 7) // 8 * 8, L // 2)
    lo = j * half
    hi = jnp.where(j == 0, half, ctx)

    pltpu.sync_copy(af_hbm.at[pl.ds(b * L, L)], idx_v)

    row0 = b * L

    def chunk_base(i):
        return jnp.minimum(lo + i * CH, L - CH)

    def body(i, _):
        s = i % 2
        pred = lo + i * CH < hi
        prevp = pred & (i >= 2)

        @pl.when(prevp)
        def _wait_prev_wb():
            @pl.when(s == 0)
            def _():
                pltpu.make_async_copy(kbuf.at[0], kg_hbm.at[pl.ds(row0, CH)], wsem0).wait()
                pltpu.make_async_copy(vbuf.at[0], vg_hbm.at[pl.ds(row0, CH)], wsem0).wait()
            @pl.when(s == 1)
            def _():
                pltpu.make_async_copy(kbuf.at[1], kg_hbm.at[pl.ds(row0, CH)], wsem1).wait()
                pltpu.make_async_copy(vbuf.at[1], vg_hbm.at[pl.ds(row0, CH)], wsem1).wait()

        @pl.when(pred)
        def _do_chunk():
            base = chunk_base(i)
            idx = idx_v.at[pl.ds(base, CH)]
            pltpu.make_async_copy(kc_hbm.at[idx], kbuf.at[s], gsem).start()
            pltpu.make_async_copy(vc_hbm.at[idx], vbuf.at[s], gsem).start()
            pltpu.make_async_copy(kc_hbm.at[idx], kbuf.at[s], gsem).wait()
            pltpu.make_async_copy(vc_hbm.at[idx], vbuf.at[s], gsem).wait()
            dstk = kg_hbm.at[pl.ds(row0 + base, CH)]
            dstv = vg_hbm.at[pl.ds(row0 + base, CH)]

            @pl.when(s == 0)
            def _():
                pltpu.make_async_copy(kbuf.at[0], dstk, wsem0).start()
                pltpu.make_async_copy(vbuf.at[0], dstv, wsem0).start()
            @pl.when(s == 1)
            def _():
                pltpu.make_async_copy(kbuf.at[1], dstk, wsem1).start()
                pltpu.make_async_copy(vbuf.at[1], dstv, wsem1).start()
        return 0

    lax.fori_loop(0, MAXCH, body, 0)

    nv = jnp.maximum((hi - lo + CH - 1) // CH, 0)

    def drain(slot_parity):
        @pl.when(slot_parity == 0)
        def _():
            pltpu.make_async_copy(kbuf.at[0], kg_hbm.at[pl.ds(row0, CH)], wsem0).wait()
            pltpu.make_async_copy(vbuf.at[0], vg_hbm.at[pl.ds(row0, CH)], wsem0).wait()
        @pl.when(slot_parity == 1)
        def _():
            pltpu.make_async_copy(kbuf.at[1], kg_hbm.at[pl.ds(row0, CH)], wsem1).wait()
            pltpu.make_async_copy(vbuf.at[1], vg_hbm.at[pl.ds(row0, CH)], wsem1).wait()

    @pl.when(nv >= 2)
    def _():
        drain((nv - 2) % 2)

    @pl.when(nv >= 1)
    def _():
        drain((nv - 1) % 2)


def _sc_gather(kc2, vc2, af, context_lens):
    fn = pl.kernel(
        _sc_gather_body,
        out_type=(jax.ShapeDtypeStruct((B * L, D), jnp.float32),
                  jax.ShapeDtypeStruct((B * L, D), jnp.float32)),
        mesh=plsc.VectorSubcoreMesh(core_axis_name="c", subcore_axis_name="s",
                                    num_cores=NC, num_subcores=NS),
        scratch_types=[
            pltpu.VMEM((L,), jnp.int32),
            pltpu.VMEM((2 * NS,), jnp.int32),
            pltpu.VMEM((2, CH, D), jnp.float32),
            pltpu.VMEM((2, CH, D), jnp.float32),
            pltpu.SemaphoreType.DMA,
            pltpu.SemaphoreType.DMA,
            pltpu.SemaphoreType.DMA,
        ],
    )
    return fn(kc2, vc2, af, context_lens)


def _attn_body(ctx_ref, q_ref, ids_ref, sm_ref, knew_ref, vnew_ref,
               kg_ref, vg_ref, o_ref, m_scr, s_scr, acc_scr):
    b = pl.program_id(0)
    c = pl.program_id(1)
    ctx = ctx_ref[b]
    nlast = (ctx + L_BLK - 1) // L_BLK - 1

    @pl.when(c == 0)
    def _init():
        m_scr[...] = jnp.full((H, 128), NEG, jnp.float32)
        s_scr[...] = jnp.zeros((H, 128), jnp.float32)
        acc_scr[...] = jnp.zeros((H, D), jnp.float32)

    @pl.when(c <= nlast)
    def _compute():
        ids = ids_ref[0, 0]
        smv = sm_ref[...]
        onehot_t = (smv == ids).astype(jnp.float32)
        dn = (((0,), (0,)), ((), ()))
        repl_k = lax.dot_general(onehot_t, knew_ref[...], dn,
                                 preferred_element_type=jnp.float32)
        repl_v = lax.dot_general(onehot_t, vnew_ref[...], dn,
                                 preferred_element_type=jnp.float32)
        anym = lax.dot_general(onehot_t, jnp.ones((16, 1), jnp.float32), dn,
                               preferred_element_type=jnp.float32)
        liota = lax.broadcasted_iota(jnp.int32, (L_BLK, 1), 0) + c * L_BLK
        kf = kg_ref[0, 0] * (1.0 - anym) + repl_k
        vf = vg_ref[0, 0] * (1.0 - anym) + repl_v
        vf = jnp.where(liota < ctx, vf, 0.0)

        qb = q_ref[0]
        logits = lax.dot_general(qb, kf, (((1,), (1,)), ((), ())),
                                 preferred_element_type=jnp.float32) * SCALE
        cmask = lax.broadcasted_iota(jnp.int32, (1, L_BLK), 1) + c * L_BLK < ctx
        logits = jnp.where(cmask, logits, NEG)

        m_prev = m_scr[:, :1]
        m_new = jnp.maximum(m_prev, jnp.max(logits, axis=1, keepdims=True))
        alpha = jnp.exp(m_prev - m_new)
        p = jnp.exp(logits - m_new)
        s_new = s_scr[:, :1] * alpha + jnp.sum(p, axis=1, keepdims=True)
        acc_scr[...] = acc_scr[...] * alpha + lax.dot_general(
            p, vf, (((1,), (0,)), ((), ())), preferred_element_type=jnp.float32)
        m_scr[...] = jnp.broadcast_to(m_new, (H, 128))
        s_scr[...] = jnp.broadcast_to(s_new, (H, 128))

    @pl.when(c == NBLK - 1)
    def _fin():
        accv = acc_scr[...] / s_scr[:, :1]
        rowh = lax.broadcasted_iota(jnp.int32, (H, 1), 0) // GROUP
        o = jnp.zeros((H, Dh), jnp.float32)
        for hh in range(KVH):
            o = o + jnp.where(rowh == hh, accv[:, hh * Dh:(hh + 1) * Dh], 0.0)
        o_ref[0] = o


def _ceff(c, ctx):
    return jnp.minimum(c, jnp.maximum((ctx + L_BLK - 1) // L_BLK - 1, 0))


def _attn(context_lens, q_bd, active4, sm2, knew, vnew, kg4, vg4):
    grid_spec = pltpu.PrefetchScalarGridSpec(
        num_scalar_prefetch=1,
        grid=(B, NBLK),
        in_specs=[
            pl.BlockSpec((1, H, D), lambda b, c, ctx: (b, 0, 0)),
            pl.BlockSpec((1, 1, 1, L_BLK),
                         lambda b, c, ctx: (b, _ceff(c, ctx[b]), 0, 0)),
            pl.BlockSpec((16, 1), lambda b, c, ctx: (0, 0)),
            pl.BlockSpec((16, D), lambda b, c, ctx: (0, 0)),
            pl.BlockSpec((16, D), lambda b, c, ctx: (0, 0)),
            pl.BlockSpec((1, 1, L_BLK, D),
                         lambda b, c, ctx: (b, _ceff(c, ctx[b]), 0, 0)),
            pl.BlockSpec((1, 1, L_BLK, D),
                         lambda b, c, ctx: (b, _ceff(c, ctx[b]), 0, 0)),
        ],
        out_specs=pl.BlockSpec((1, H, Dh), lambda b, c, ctx: (b, 0, 0)),
        scratch_shapes=[
            pltpu.VMEM((H, 128), jnp.float32),
            pltpu.VMEM((H, 128), jnp.float32),
            pltpu.VMEM((H, D), jnp.float32),
        ],
    )
    return pl.pallas_call(
        _attn_body,
        grid_spec=grid_spec,
        out_shape=jax.ShapeDtypeStruct((B, H, Dh), jnp.float32),
        compiler_params=pltpu.CompilerParams(
            dimension_semantics=("arbitrary", "arbitrary")),
    )(context_lens, q_bd, active4, sm2, knew, vnew, kg4, vg4)


def _build_q_bd(q):
    q_tiled = jnp.tile(q, (1, 1, KVH))
    rowh = jnp.arange(H) // GROUP
    colh = jnp.arange(D) // Dh
    mask = (rowh[:, None] == colh[None, :]).astype(q.dtype)
    return q_tiled * mask[None]


def kernel(q, k, v, k_cache, v_cache, slot_mapping, active_slots, context_lens):
    kc2 = k_cache.reshape(NUM_SLOTS, D)
    vc2 = v_cache.reshape(NUM_SLOTS, D)
    af = active_slots.reshape(B * L)
    kg, vg = _sc_gather(kc2, vc2, af, context_lens)

    q_bd = _build_q_bd(q)
    active4 = active_slots.reshape(B, NBLK, 1, L_BLK)
    sm2 = slot_mapping.reshape(16, 1)
    knew = k.reshape(B, D)
    vnew = v.reshape(B, D)
    kg4 = kg.reshape(B, NBLK, L_BLK, D)
    vg4 = vg.reshape(B, NBLK, L_BLK, D)
    return _attn(context_lens, q_bd, active4, sm2, knew, vnew, kg4, vg4)


# speedup vs baseline: 2.8366x; 1.0815x over previous
import functools

import jax
import jax.numpy as jnp
from jax import lax
from jax.experimental import pallas as pl
from jax.experimental.pallas import tpu as pltpu
from jax.experimental.pallas import tpu_sc as plsc

B, H, KVH, Dh = 16, 32, 8, 128
NUM_SLOTS, L = 32768, 2048
SCALE = 0.08838834764831845
GROUP = H // KVH
D = KVH * Dh
NEG = -1e30

NC, NS = 2, 16
NW = NC * NS
CH = 24
MAXCH = (L // 2 + CH - 1) // CH + 1

L_BLK = 256
NBLK = L // L_BLK


def _sc_gather_body(kc_hbm, vc_hbm, af_hbm, ctx_hbm, kg_hbm, vg_hbm,
                    idx_v, ctx_v, kbuf, vbuf, gsem, wsem0, wsem1):
    cid = lax.axis_index("c")
    sid = lax.axis_index("s")
    wid = sid * NC + cid
    b = wid // 2
    j = wid % 2

    pltpu.sync_copy(ctx_hbm, ctx_v.at[pl.ds(0, NS)])
    ctx = ctx_v[pl.ds(b, NS)][0]

    half = jnp.minimum(((ctx + 1) // 2 + 7) // 8 * 8, L // 2)
    lo = j * half
    hi = jnp.where(j == 0, half, ctx)

    pltpu.sync_copy(af_hbm.at[pl.ds(b * L, L)], idx_v)

    row0 = b * L

    def chunk_base(i):
        return jnp.minimum(lo + i * CH, L - CH)

    def body(i, _):
        s = i % 2
        pred = lo + i * CH < hi
        prevp = pred & (i >= 2)

        @pl.when(prevp)
        def _wait_prev_wb():
            @pl.when(s == 0)
            def _():
                pltpu.make_async_copy(kbuf.at[0], kg_hbm.at[pl.ds(row0, CH)], wsem0).wait()
                pltpu.make_async_copy(vbuf.at[0], vg_hbm.at[pl.ds(row0, CH)], wsem0).wait()
            @pl.when(s == 1)
            def _():
                pltpu.make_async_copy(kbuf.at[1], kg_hbm.at[pl.ds(row0, CH)], wsem1).wait()
                pltpu.make_async_copy(vbuf.at[1], vg_hbm.at[pl.ds(row0, CH)], wsem1).wait()

        @pl.when(pred)
        def _do_chunk():
            base = chunk_base(i)
            idx = idx_v.at[pl.ds(base, CH)]
            pltpu.make_async_copy(kc_hbm.at[idx], kbuf.at[s], gsem).start()
            pltpu.make_async_copy(vc_hbm.at[idx], vbuf.at[s], gsem).start()
            pltpu.make_async_copy(kc_hbm.at[idx], kbuf.at[s], gsem).wait()
            pltpu.make_async_copy(vc_hbm.at[idx], vbuf.at[s], gsem).wait()
            dstk = kg_hbm.at[pl.ds(row0 + base, CH)]
            dstv = vg_hbm.at[pl.ds(row0 + base, CH)]

            @pl.when(s == 0)
            def _():
                pltpu.make_async_copy(kbuf.at[0], dstk, wsem0).start()
                pltpu.make_async_copy(vbuf.at[0], dstv, wsem0).start()
            @pl.when(s == 1)
            def _():
                pltpu.make_async_copy(kbuf.at[1], dstk, wsem1).start()
                pltpu.make_async_copy(vbuf.at[1], dstv, wsem1).start()
        return 0

    lax.fori_loop(0, MAXCH, body, 0)

    nv = jnp.maximum((hi - lo + CH - 1) // CH, 0)

    def drain(slot_parity):
        @pl.when(slot_parity == 0)
        def _():
            pltpu.make_async_copy(kbuf.at[0], kg_hbm.at[pl.ds(row0, CH)], wsem0).wait()
            pltpu.make_async_copy(vbuf.at[0], vg_hbm.at[pl.ds(row0, CH)], wsem0).wait()
        @pl.when(slot_parity == 1)
        def _():
            pltpu.make_async_copy(kbuf.at[1], kg_hbm.at[pl.ds(row0, CH)], wsem1).wait()
            pltpu.make_async_copy(vbuf.at[1], vg_hbm.at[pl.ds(row0, CH)], wsem1).wait()

    @pl.when(nv >= 2)
    def _():
        drain((nv - 2) % 2)

    @pl.when(nv >= 1)
    def _():
        drain((nv - 1) % 2)


def _sc_gather(kc2, vc2, af, context_lens):
    fn = pl.kernel(
        _sc_gather_body,
        out_type=(jax.ShapeDtypeStruct((B * L, D), jnp.float32),
                  jax.ShapeDtypeStruct((B * L, D), jnp.float32)),
        mesh=plsc.VectorSubcoreMesh(core_axis_name="c", subcore_axis_name="s",
                                    num_cores=NC, num_subcores=NS),
        scratch_types=[
            pltpu.VMEM((L,), jnp.int32),
            pltpu.VMEM((2 * NS,), jnp.int32),
            pltpu.VMEM((2, CH, D), jnp.float32),
            pltpu.VMEM((2, CH, D), jnp.float32),
            pltpu.SemaphoreType.DMA,
            pltpu.SemaphoreType.DMA,
            pltpu.SemaphoreType.DMA,
        ],
    )
    return fn(kc2, vc2, af, context_lens)


def _attn_body(ctx_ref, q_ref, ids_ref, sm_ref, knew_ref, vnew_ref,
               kg_ref, vg_ref, o_ref, m_scr, s_scr, acc_scr, qk_scr):
    b = pl.program_id(0)
    c = pl.program_id(1)
    ctx = ctx_ref[b]
    nlast = (ctx + L_BLK - 1) // L_BLK - 1

    @pl.when(c == 0)
    def _init():
        m_scr[...] = jnp.full((H, 128), NEG, jnp.float32)
        s_scr[...] = jnp.zeros((H, 128), jnp.float32)
        acc_scr[...] = jnp.zeros((H, D), jnp.float32)
        qk_scr[...] = lax.dot_general(q_ref[0], knew_ref[...],
                                      (((1,), (1,)), ((), ())),
                                      preferred_element_type=jnp.float32)

    @pl.when(c <= nlast)
    def _compute():
        ids = ids_ref[0, 0]
        smv = sm_ref[...]
        onehot_t = (smv == ids).astype(jnp.float32)
        any_row = jnp.max(onehot_t, axis=0, keepdims=True)

        qb = q_ref[0]
        raw = lax.dot_general(qb, kg_ref[0, 0], (((1,), (1,)), ((), ())),
                              preferred_element_type=jnp.float32)
        sel = lax.dot_general(qk_scr[...], onehot_t, (((1,), (0,)), ((), ())),
                              preferred_element_type=jnp.float32)
        logits = (raw * (1.0 - any_row) + sel) * SCALE
        cmask = lax.broadcasted_iota(jnp.int32, (1, L_BLK), 1) + c * L_BLK < ctx
        logits = jnp.where(cmask, logits, NEG)

        m_prev = m_scr[:, :1]
        m_new = jnp.maximum(m_prev, jnp.max(logits, axis=1, keepdims=True))
        alpha = jnp.exp(m_prev - m_new)
        p = jnp.exp(logits - m_new)
        s_new = s_scr[:, :1] * alpha + jnp.sum(p, axis=1, keepdims=True)
        m_scr[...] = jnp.broadcast_to(m_new, (H, 128))
        s_scr[...] = jnp.broadcast_to(s_new, (H, 128))

        pm = p * (1.0 - any_row)
        pvj = lax.dot_general(p, onehot_t, (((1,), (1,)), ((), ())),
                              preferred_element_type=jnp.float32)
        accn = lax.dot_general(pvj, vnew_ref[...], (((1,), (0,)), ((), ())),
                               preferred_element_type=jnp.float32)

        @pl.when(c < nlast)
        def _pv_full():
            acc_scr[...] = acc_scr[...] * alpha + accn + lax.dot_general(
                pm, vg_ref[0, 0], (((1,), (0,)), ((), ())),
                preferred_element_type=jnp.float32)

        @pl.when(c == nlast)
        def _pv_straddle():
            liota = lax.broadcasted_iota(jnp.int32, (L_BLK, 1), 0) + c * L_BLK
            vgm = jnp.where(liota < ctx, vg_ref[0, 0], 0.0)
            acc_scr[...] = acc_scr[...] * alpha + accn + lax.dot_general(
                pm, vgm, (((1,), (0,)), ((), ())),
                preferred_element_type=jnp.float32)

    @pl.when(c == NBLK - 1)
    def _fin():
        accv = acc_scr[...] / s_scr[:, :1]
        rowh = lax.broadcasted_iota(jnp.int32, (H, 1), 0) // GROUP
        o = jnp.zeros((H, Dh), jnp.float32)
        for hh in range(KVH):
            o = o + jnp.where(rowh == hh, accv[:, hh * Dh:(hh + 1) * Dh], 0.0)
        o_ref[0] = o


def _ceff(c, ctx):
    return jnp.minimum(c, jnp.maximum((ctx + L_BLK - 1) // L_BLK - 1, 0))


def _attn(context_lens, q_bd, active4, sm2, knew, vnew, kg4, vg4):
    grid_spec = pltpu.PrefetchScalarGridSpec(
        num_scalar_prefetch=1,
        grid=(B, NBLK),
        in_specs=[
            pl.BlockSpec((1, H, D), lambda b, c, ctx: (b, 0, 0)),
            pl.BlockSpec((1, 1, 1, L_BLK),
                         lambda b, c, ctx: (b, _ceff(c, ctx[b]), 0, 0)),
            pl.BlockSpec((16, 1), lambda b, c, ctx: (0, 0)),
            pl.BlockSpec((16, D), lambda b, c, ctx: (0, 0)),
            pl.BlockSpec((16, D), lambda b, c, ctx: (0, 0)),
            pl.BlockSpec((1, 1, L_BLK, D),
                         lambda b, c, ctx: (b, _ceff(c, ctx[b]), 0, 0)),
            pl.BlockSpec((1, 1, L_BLK, D),
                         lambda b, c, ctx: (b, _ceff(c, ctx[b]), 0, 0)),
        ],
        out_specs=pl.BlockSpec((1, H, Dh), lambda b, c, ctx: (b, 0, 0)),
        scratch_shapes=[
            pltpu.VMEM((H, 128), jnp.float32),
            pltpu.VMEM((H, 128), jnp.float32),
            pltpu.VMEM((H, D), jnp.float32),
            pltpu.VMEM((H, 16), jnp.float32),
        ],
    )
    return pl.pallas_call(
        _attn_body,
        grid_spec=grid_spec,
        out_shape=jax.ShapeDtypeStruct((B, H, Dh), jnp.float32),
        compiler_params=pltpu.CompilerParams(
            dimension_semantics=("arbitrary", "arbitrary")),
    )(context_lens, q_bd, active4, sm2, knew, vnew, kg4, vg4)


def _build_q_bd(q):
    q_tiled = jnp.tile(q, (1, 1, KVH))
    rowh = jnp.arange(H) // GROUP
    colh = jnp.arange(D) // Dh
    mask = (rowh[:, None] == colh[None, :]).astype(q.dtype)
    return q_tiled * mask[None]


def kernel(q, k, v, k_cache, v_cache, slot_mapping, active_slots, context_lens):
    kc2 = k_cache.reshape(NUM_SLOTS, D)
    vc2 = v_cache.reshape(NUM_SLOTS, D)
    af = active_slots.reshape(B * L)
    kg, vg = _sc_gather(kc2, vc2, af, context_lens)

    q_bd = _build_q_bd(q)
    active4 = active_slots.reshape(B, NBLK, 1, L_BLK)
    sm2 = slot_mapping.reshape(16, 1)
    knew = k.reshape(B, D)
    vnew = v.reshape(B, D)
    kg4 = kg.reshape(B, NBLK, L_BLK, D)
    vg4 = vg.reshape(B, NBLK, L_BLK, D)
    return _attn(context_lens, q_bd, active4, sm2, knew, vnew, kg4, vg4)


# speedup vs baseline: 3.0781x; 1.0851x over previous
import functools

import jax
import jax.numpy as jnp
from jax import lax
from jax.experimental import pallas as pl
from jax.experimental.pallas import tpu as pltpu
from jax.experimental.pallas import tpu_sc as plsc

B, H, KVH, Dh = 16, 32, 8, 128
NUM_SLOTS, L = 32768, 2048
SCALE = 0.08838834764831845
GROUP = H // KVH
D = KVH * Dh
NEG = -1e30

NC, NS = 2, 16
NW = NC * NS
CH = 24
MAXCH = (L // 2 + CH - 1) // CH + 1

L_BLK = 512
NBLK = L // L_BLK


def _sc_gather_body(kc_hbm, vc_hbm, af_hbm, ctx_hbm, kg_hbm, vg_hbm,
                    idx_v, ctx_v, kbuf, vbuf, gsem, wsem0, wsem1):
    cid = lax.axis_index("c")
    sid = lax.axis_index("s")
    wid = sid * NC + cid
    b = wid // 2
    j = wid % 2

    pltpu.sync_copy(ctx_hbm, ctx_v.at[pl.ds(0, NS)])
    ctx = ctx_v[pl.ds(b, NS)][0]

    half = jnp.minimum(((ctx + 1) // 2 + 7) // 8 * 8, L // 2)
    lo = j * half
    hi = jnp.where(j == 0, half, ctx)

    pltpu.sync_copy(af_hbm.at[pl.ds(b * L, L)], idx_v)

    row0 = b * L

    def chunk_base(i):
        return jnp.minimum(lo + i * CH, L - CH)

    def body(i, _):
        s = i % 2
        pred = lo + i * CH < hi
        prevp = pred & (i >= 2)

        @pl.when(prevp)
        def _wait_prev_wb():
            @pl.when(s == 0)
            def _():
                pltpu.make_async_copy(kbuf.at[0], kg_hbm.at[pl.ds(row0, CH)], wsem0).wait()
                pltpu.make_async_copy(vbuf.at[0], vg_hbm.at[pl.ds(row0, CH)], wsem0).wait()
            @pl.when(s == 1)
            def _():
                pltpu.make_async_copy(kbuf.at[1], kg_hbm.at[pl.ds(row0, CH)], wsem1).wait()
                pltpu.make_async_copy(vbuf.at[1], vg_hbm.at[pl.ds(row0, CH)], wsem1).wait()

        @pl.when(pred)
        def _do_chunk():
            base = chunk_base(i)
            idx = idx_v.at[pl.ds(base, CH)]
            pltpu.make_async_copy(kc_hbm.at[idx], kbuf.at[s], gsem).start()
            pltpu.make_async_copy(vc_hbm.at[idx], vbuf.at[s], gsem).start()
            pltpu.make_async_copy(kc_hbm.at[idx], kbuf.at[s], gsem).wait()
            pltpu.make_async_copy(vc_hbm.at[idx], vbuf.at[s], gsem).wait()
            dstk = kg_hbm.at[pl.ds(row0 + base, CH)]
            dstv = vg_hbm.at[pl.ds(row0 + base, CH)]

            @pl.when(s == 0)
            def _():
                pltpu.make_async_copy(kbuf.at[0], dstk, wsem0).start()
                pltpu.make_async_copy(vbuf.at[0], dstv, wsem0).start()
            @pl.when(s == 1)
            def _():
                pltpu.make_async_copy(kbuf.at[1], dstk, wsem1).start()
                pltpu.make_async_copy(vbuf.at[1], dstv, wsem1).start()
        return 0

    lax.fori_loop(0, MAXCH, body, 0)

    nv = jnp.maximum((hi - lo + CH - 1) // CH, 0)

    def drain(slot_parity):
        @pl.when(slot_parity == 0)
        def _():
            pltpu.make_async_copy(kbuf.at[0], kg_hbm.at[pl.ds(row0, CH)], wsem0).wait()
            pltpu.make_async_copy(vbuf.at[0], vg_hbm.at[pl.ds(row0, CH)], wsem0).wait()
        @pl.when(slot_parity == 1)
        def _():
            pltpu.make_async_copy(kbuf.at[1], kg_hbm.at[pl.ds(row0, CH)], wsem1).wait()
            pltpu.make_async_copy(vbuf.at[1], vg_hbm.at[pl.ds(row0, CH)], wsem1).wait()

    @pl.when(nv >= 2)
    def _():
        drain((nv - 2) % 2)

    @pl.when(nv >= 1)
    def _():
        drain((nv - 1) % 2)


def _sc_gather(kc2, vc2, af, context_lens):
    fn = pl.kernel(
        _sc_gather_body,
        out_type=(jax.ShapeDtypeStruct((B * L, D), jnp.float32),
                  jax.ShapeDtypeStruct((B * L, D), jnp.float32)),
        mesh=plsc.VectorSubcoreMesh(core_axis_name="c", subcore_axis_name="s",
                                    num_cores=NC, num_subcores=NS),
        scratch_types=[
            pltpu.VMEM((L,), jnp.int32),
            pltpu.VMEM((2 * NS,), jnp.int32),
            pltpu.VMEM((2, CH, D), jnp.float32),
            pltpu.VMEM((2, CH, D), jnp.float32),
            pltpu.SemaphoreType.DMA,
            pltpu.SemaphoreType.DMA,
            pltpu.SemaphoreType.DMA,
        ],
    )
    return fn(kc2, vc2, af, context_lens)


def _attn_body(ctx_ref, q_ref, ids_ref, sm_ref, knew_ref, vnew_ref,
               kg_ref, vg_ref, o_ref, m_scr, s_scr, acc_scr, qk_scr):
    b = pl.program_id(0)
    c = pl.program_id(1)
    ctx = ctx_ref[b]
    nlast = (ctx + L_BLK - 1) // L_BLK - 1

    @pl.when(c == 0)
    def _init():
        m_scr[...] = jnp.full((H, 128), NEG, jnp.float32)
        s_scr[...] = jnp.zeros((H, 128), jnp.float32)
        acc_scr[...] = jnp.zeros((H, D), jnp.float32)
        qk_scr[...] = lax.dot_general(q_ref[0], knew_ref[...],
                                      (((1,), (1,)), ((), ())),
                                      preferred_element_type=jnp.float32)

    @pl.when(c <= nlast)
    def _compute():
        ids = ids_ref[0, 0]
        smv = sm_ref[...]
        onehot_t = (smv == ids).astype(jnp.float32)
        any_row = jnp.max(onehot_t, axis=0, keepdims=True)

        qb = q_ref[0]
        raw = lax.dot_general(qb, kg_ref[0, 0], (((1,), (1,)), ((), ())),
                              preferred_element_type=jnp.float32)
        sel = lax.dot_general(qk_scr[...], onehot_t, (((1,), (0,)), ((), ())),
                              preferred_element_type=jnp.float32)
        logits = (raw * (1.0 - any_row) + sel) * SCALE
        cmask = lax.broadcasted_iota(jnp.int32, (1, L_BLK), 1) + c * L_BLK < ctx
        logits = jnp.where(cmask, logits, NEG)

        m_prev = m_scr[:, :1]
        m_new = jnp.maximum(m_prev, jnp.max(logits, axis=1, keepdims=True))
        alpha = jnp.exp(m_prev - m_new)
        p = jnp.exp(logits - m_new)
        s_new = s_scr[:, :1] * alpha + jnp.sum(p, axis=1, keepdims=True)
        m_scr[...] = jnp.broadcast_to(m_new, (H, 128))
        s_scr[...] = jnp.broadcast_to(s_new, (H, 128))

        pm = p * (1.0 - any_row)
        pvj = lax.dot_general(p, onehot_t, (((1,), (1,)), ((), ())),
                              preferred_element_type=jnp.float32)
        accn = lax.dot_general(pvj, vnew_ref[...], (((1,), (0,)), ((), ())),
                               preferred_element_type=jnp.float32)

        @pl.when(c < nlast)
        def _pv_full():
            acc_scr[...] = acc_scr[...] * alpha + accn + lax.dot_general(
                pm, vg_ref[0, 0], (((1,), (0,)), ((), ())),
                preferred_element_type=jnp.float32)

        @pl.when(c == nlast)
        def _pv_straddle():
            liota = lax.broadcasted_iota(jnp.int32, (L_BLK, 1), 0) + c * L_BLK
            vgm = jnp.where(liota < ctx, vg_ref[0, 0], 0.0)
            acc_scr[...] = acc_scr[...] * alpha + accn + lax.dot_general(
                pm, vgm, (((1,), (0,)), ((), ())),
                preferred_element_type=jnp.float32)

    @pl.when(c == NBLK - 1)
    def _fin():
        accv = acc_scr[...] / s_scr[:, :1]
        rowh = lax.broadcasted_iota(jnp.int32, (H, 1), 0) // GROUP
        o = jnp.zeros((H, Dh), jnp.float32)
        for hh in range(KVH):
            o = o + jnp.where(rowh == hh, accv[:, hh * Dh:(hh + 1) * Dh], 0.0)
        o_ref[0] = o


def _ceff(c, ctx):
    return jnp.minimum(c, jnp.maximum((ctx + L_BLK - 1) // L_BLK - 1, 0))


def _attn(context_lens, q_bd, active4, sm2, knew, vnew, kg4, vg4):
    grid_spec = pltpu.PrefetchScalarGridSpec(
        num_scalar_prefetch=1,
        grid=(B, NBLK),
        in_specs=[
            pl.BlockSpec((1, H, D), lambda b, c, ctx: (b, 0, 0)),
            pl.BlockSpec((1, 1, 1, L_BLK),
                         lambda b, c, ctx: (b, _ceff(c, ctx[b]), 0, 0)),
            pl.BlockSpec((16, 1), lambda b, c, ctx: (0, 0)),
            pl.BlockSpec((16, D), lambda b, c, ctx: (0, 0)),
            pl.BlockSpec((16, D), lambda b, c, ctx: (0, 0)),
            pl.BlockSpec((1, 1, L_BLK, D),
                         lambda b, c, ctx: (b, _ceff(c, ctx[b]), 0, 0)),
            pl.BlockSpec((1, 1, L_BLK, D),
                         lambda b, c, ctx: (b, _ceff(c, ctx[b]), 0, 0)),
        ],
        out_specs=pl.BlockSpec((1, H, Dh), lambda b, c, ctx: (b, 0, 0)),
        scratch_shapes=[
            pltpu.VMEM((H, 128), jnp.float32),
            pltpu.VMEM((H, 128), jnp.float32),
            pltpu.VMEM((H, D), jnp.float32),
            pltpu.VMEM((H, 16), jnp.float32),
        ],
    )
    return pl.pallas_call(
        _attn_body,
        grid_spec=grid_spec,
        out_shape=jax.ShapeDtypeStruct((B, H, Dh), jnp.float32),
        compiler_params=pltpu.CompilerParams(
            dimension_semantics=("arbitrary", "arbitrary")),
    )(context_lens, q_bd, active4, sm2, knew, vnew, kg4, vg4)


def _build_q_bd(q):
    q_tiled = jnp.tile(q, (1, 1, KVH))
    rowh = jnp.arange(H) // GROUP
    colh = jnp.arange(D) // Dh
    mask = (rowh[:, None] == colh[None, :]).astype(q.dtype)
    return q_tiled * mask[None]


def kernel(q, k, v, k_cache, v_cache, slot_mapping, active_slots, context_lens):
    kc2 = k_cache.reshape(NUM_SLOTS, D)
    vc2 = v_cache.reshape(NUM_SLOTS, D)
    af = active_slots.reshape(B * L)
    kg, vg = _sc_gather(kc2, vc2, af, context_lens)

    q_bd = _build_q_bd(q)
    active4 = active_slots.reshape(B, NBLK, 1, L_BLK)
    sm2 = slot_mapping.reshape(16, 1)
    knew = k.reshape(B, D)
    vnew = v.reshape(B, D)
    kg4 = kg.reshape(B, NBLK, L_BLK, D)
    vg4 = vg.reshape(B, NBLK, L_BLK, D)
    return _attn(context_lens, q_bd, active4, sm2, knew, vnew, kg4, vg4)
